# baseline (speedup 1.0000x reference)
import functools

import jax
import jax.numpy as jnp
import numpy as np
from jax import lax
from jax.experimental import pallas as pl
from jax.experimental.pallas import tpu as pltpu
from jax.experimental.pallas import tpu_sc as plsc

B = 128
V = 100000
VC = 2000
NB = V // VC
UNROLL = 2
UNROLL2 = 4

_K1 = np.uint32(0)
_K2 = np.uint32(42)
_TINY = np.float32(1.1754943508222875e-38)
_I32_BIG = np.int32(2**31 - 1)


def _threefry_bits(cnt_lo):
    ks0 = _K1
    ks1 = _K2
    ks2 = np.uint32(_K1 ^ _K2 ^ np.uint32(0x1BD11BDA))

    x0 = jnp.zeros_like(cnt_lo) + ks0
    x1 = cnt_lo + ks1

    def rnd(x0, x1, r):
        x0 = x0 + x1
        x1 = (x1 << np.uint32(r)) | (x1 >> np.uint32(32 - r))
        x1 = x0 ^ x1
        return x0, x1

    for r in (13, 15, 26, 6):
        x0, x1 = rnd(x0, x1, r)
    x0 = x0 + ks1
    x1 = x1 + np.uint32(ks2 + np.uint32(1))
    for r in (17, 29, 16, 24):
        x0, x1 = rnd(x0, x1, r)
    x0 = x0 + ks2
    x1 = x1 + np.uint32(ks0 + np.uint32(2))
    for r in (13, 15, 26, 6):
        x0, x1 = rnd(x0, x1, r)
    x0 = x0 + ks0
    x1 = x1 + np.uint32(ks1 + np.uint32(3))
    for r in (17, 29, 16, 24):
        x0, x1 = rnd(x0, x1, r)
    x0 = x0 + ks1
    x1 = x1 + np.uint32(ks2 + np.uint32(4))
    for r in (13, 15, 26, 6):
        x0, x1 = rnd(x0, x1, r)
    x0 = x0 + ks2
    x1 = x1 + np.uint32(ks0 + np.uint32(5))
    return x0 ^ x1


BR = 16
NR = B // BR


def _dense_body(logits_ref, lp_ref, idx_ref):
    j = pl.program_id(0)
    row0 = j * BR

    row = lax.broadcasted_iota(jnp.int32, (BR, VC), 0) + row0
    col0 = lax.broadcasted_iota(jnp.int32, (BR, VC), 1)
    cnt0 = row * V + col0

    def chunk(k, carry):
        a_score, a_cnt = carry
        blk = logits_ref[:, k, :]
        cnt = cnt0 + k * VC

        bits = _threefry_bits(cnt.astype(jnp.uint32))
        fbits = (bits >> np.uint32(9)) | np.uint32(0x3F800000)
        floats = lax.bitcast_convert_type(fbits, jnp.float32) - np.float32(1.0)
        u = jnp.maximum(_TINY, floats + _TINY)
        score = blk - jnp.log(-jnp.log(u))

        upd = score > a_score
        a_score = jnp.where(upd, score, a_score)
        a_cnt = jnp.where(upd, cnt, a_cnt)
        return a_score, a_cnt

    neg_inf = jnp.full((BR, VC), -jnp.inf, jnp.float32)
    a_score, a_cnt = lax.fori_loop(
        0, NB, chunk, (neg_inf, jnp.zeros((BR, VC), jnp.int32)),
        unroll=UNROLL)

    def chunk2a(k, a_lmax):
        return jnp.maximum(a_lmax, logits_ref[:, k, :])

    a_lmax = lax.fori_loop(0, NB, chunk2a, neg_inf, unroll=UNROLL2)

    def chunk2b(k, se):
        return se + jnp.exp(logits_ref[:, k, :] - a_lmax)

    se = lax.fori_loop(0, NB, chunk2b, jnp.zeros((BR, VC), jnp.float32),
                       unroll=UNROLL2)

    m_row = jnp.max(a_lmax, axis=1)
    se_row = jnp.sum(se * jnp.exp(a_lmax - m_row[:, None]), axis=1)

    m_sc = jnp.max(a_score, axis=1)
    cand = jnp.where(a_score == m_sc[:, None], a_cnt, _I32_BIG)
    cnt_win = jnp.min(cand, axis=1)

    bits_w = _threefry_bits(cnt_win.astype(jnp.uint32))
    fb_w = (bits_w >> np.uint32(9)) | np.uint32(0x3F800000)
    fl_w = lax.bitcast_convert_type(fb_w, jnp.float32) - np.float32(1.0)
    u_w = jnp.maximum(_TINY, fl_w + _TINY)
    l_win = m_sc + jnp.log(-jnp.log(u_w))

    rvec = cnt0[:, 0]
    lp_ref[j, :] = l_win - (m_row + jnp.log(se_row))
    idx_ref[j, :] = cnt_win - rvec


def _dense_call(logits, interpret=False):
    lp2, idx2 = pl.pallas_call(
        _dense_body,
        grid=(NR,),
        in_specs=[pl.BlockSpec((BR, NB, VC), lambda j: (j, 0, 0))],
        out_specs=[pl.BlockSpec((NR, BR), lambda j: (0, 0)),
                   pl.BlockSpec((NR, BR), lambda j: (0, 0))],
        out_shape=[jax.ShapeDtypeStruct((NR, BR), jnp.float32),
                   jax.ShapeDtypeStruct((NR, BR), jnp.int32)],
        compiler_params=pltpu.CompilerParams(
            dimension_semantics=("parallel",)),
        interpret=interpret,
    )(logits.reshape(B, NB, VC))
    return lp2.reshape(B), idx2.reshape(B)


def _sc_gather(action_space, idx):
    info = plsc.get_sparse_core_info()
    nc = info.num_cores
    n_workers = 16
    per_w = B // n_workers
    mesh = plsc.VectorSubcoreMesh(core_axis_name="c", subcore_axis_name="s")

    @functools.partial(
        pl.kernel, mesh=mesh,
        out_type=jax.ShapeDtypeStruct((B,), jnp.int32),
        scratch_types=[pltpu.VMEM((per_w,), jnp.int32),
                       pltpu.VMEM((per_w,), jnp.int32),
                       pltpu.SemaphoreType.DMA],
    )
    def gather_kernel(table_hbm, idx_hbm, out_hbm, idx_v, rows_v, sem):
        wid = lax.axis_index("s") * nc + lax.axis_index("c")

        @pl.when(wid < n_workers)
        def _():
            base = wid * per_w
            pltpu.sync_copy(idx_hbm.at[pl.ds(base, per_w)], idx_v)
            pltpu.async_copy(table_hbm.at[idx_v], rows_v, sem).wait()
            pltpu.sync_copy(rows_v, out_hbm.at[pl.ds(base, per_w)])

    return gather_kernel(action_space, idx)


def kernel(logits, action_space):
    log_probs, idx = _dense_call(logits)
    actions = _sc_gather(action_space, idx)
    return log_probs, actions


# speedup vs baseline: 1.1382x; 1.1382x over previous
import functools

import jax
import jax.numpy as jnp
import numpy as np
from jax import lax
from jax.experimental import pallas as pl
from jax.experimental.pallas import tpu as pltpu
from jax.experimental.pallas import tpu_sc as plsc

B = 128
V = 100000
VC = 1000
NB = V // VC
UNROLL = 2
UNROLL2 = 4

_K1 = np.uint32(0)
_K2 = np.uint32(42)
_TINY = np.float32(1.1754943508222875e-38)
_I32_BIG = np.int32(2**31 - 1)


def _threefry_bits(cnt_lo):
    ks0 = _K1
    ks1 = _K2
    ks2 = np.uint32(_K1 ^ _K2 ^ np.uint32(0x1BD11BDA))

    x0 = jnp.zeros_like(cnt_lo) + ks0
    x1 = cnt_lo + ks1

    def rnd(x0, x1, r):
        x0 = x0 + x1
        x1 = (x1 << np.uint32(r)) | (x1 >> np.uint32(32 - r))
        x1 = x0 ^ x1
        return x0, x1

    for r in (13, 15, 26, 6):
        x0, x1 = rnd(x0, x1, r)
    x0 = x0 + ks1
    x1 = x1 + np.uint32(ks2 + np.uint32(1))
    for r in (17, 29, 16, 24):
        x0, x1 = rnd(x0, x1, r)
    x0 = x0 + ks2
    x1 = x1 + np.uint32(ks0 + np.uint32(2))
    for r in (13, 15, 26, 6):
        x0, x1 = rnd(x0, x1, r)
    x0 = x0 + ks0
    x1 = x1 + np.uint32(ks1 + np.uint32(3))
    for r in (17, 29, 16, 24):
        x0, x1 = rnd(x0, x1, r)
    x0 = x0 + ks1
    x1 = x1 + np.uint32(ks2 + np.uint32(4))
    for r in (13, 15, 26, 6):
        x0, x1 = rnd(x0, x1, r)
    x0 = x0 + ks2
    x1 = x1 + np.uint32(ks0 + np.uint32(5))
    return x0 ^ x1


BR = 8
NR = B // BR


def _dense_body(logits_ref, lp_ref, idx_ref):
    j = pl.program_id(0)
    row0 = j * BR

    row = lax.broadcasted_iota(jnp.int32, (BR, VC), 0) + row0
    col0 = lax.broadcasted_iota(jnp.int32, (BR, VC), 1)
    cnt0 = row * V + col0

    def chunk(k, carry):
        a_score, a_cnt = carry
        blk = logits_ref[:, k, :]
        cnt = cnt0 + k * VC

        bits = _threefry_bits(cnt.astype(jnp.uint32))
        fbits = (bits >> np.uint32(9)) | np.uint32(0x3F800000)
        floats = lax.bitcast_convert_type(fbits, jnp.float32) - np.float32(1.0)
        u = jnp.maximum(_TINY, floats + _TINY)
        score = blk - jnp.log(-jnp.log(u))

        upd = score > a_score
        a_score = jnp.where(upd, score, a_score)
        a_cnt = jnp.where(upd, cnt, a_cnt)
        return a_score, a_cnt

    neg_inf = jnp.full((BR, VC), -jnp.inf, jnp.float32)
    a_score, a_cnt = lax.fori_loop(
        0, NB, chunk, (neg_inf, jnp.zeros((BR, VC), jnp.int32)),
        unroll=UNROLL)

    def chunk2a(k, a_lmax):
        return jnp.maximum(a_lmax, logits_ref[:, k, :])

    a_lmax = lax.fori_loop(0, NB, chunk2a, neg_inf, unroll=UNROLL2)

    def chunk2b(k, se):
        return se + jnp.exp(logits_ref[:, k, :] - a_lmax)

    se = lax.fori_loop(0, NB, chunk2b, jnp.zeros((BR, VC), jnp.float32),
                       unroll=UNROLL2)

    m_row = jnp.max(a_lmax, axis=1)
    se_row = jnp.sum(se * jnp.exp(a_lmax - m_row[:, None]), axis=1)

    m_sc = jnp.max(a_score, axis=1)
    cand = jnp.where(a_score == m_sc[:, None], a_cnt, _I32_BIG)
    cnt_win = jnp.min(cand, axis=1)

    bits_w = _threefry_bits(cnt_win.astype(jnp.uint32))
    fb_w = (bits_w >> np.uint32(9)) | np.uint32(0x3F800000)
    fl_w = lax.bitcast_convert_type(fb_w, jnp.float32) - np.float32(1.0)
    u_w = jnp.maximum(_TINY, fl_w + _TINY)
    l_win = m_sc + jnp.log(-jnp.log(u_w))

    rvec = cnt0[:, 0]
    lp_ref[j, :] = l_win - (m_row + jnp.log(se_row))
    idx_ref[j, :] = cnt_win - rvec


def _dense_call(logits, interpret=False):
    lp2, idx2 = pl.pallas_call(
        _dense_body,
        grid=(NR,),
        in_specs=[pl.BlockSpec((BR, NB, VC), lambda j: (j, 0, 0))],
        out_specs=[pl.BlockSpec((NR, BR), lambda j: (0, 0)),
                   pl.BlockSpec((NR, BR), lambda j: (0, 0))],
        out_shape=[jax.ShapeDtypeStruct((NR, BR), jnp.float32),
                   jax.ShapeDtypeStruct((NR, BR), jnp.int32)],
        compiler_params=pltpu.CompilerParams(
            dimension_semantics=("parallel",)),
        interpret=interpret,
    )(logits.reshape(B, NB, VC))
    return lp2.reshape(B), idx2.reshape(B)


def _sc_gather(action_space, idx):
    info = plsc.get_sparse_core_info()
    nc = info.num_cores
    n_workers = 16
    per_w = B // n_workers
    mesh = plsc.VectorSubcoreMesh(core_axis_name="c", subcore_axis_name="s")

    @functools.partial(
        pl.kernel, mesh=mesh,
        out_type=jax.ShapeDtypeStruct((B,), jnp.int32),
        scratch_types=[pltpu.VMEM((per_w,), jnp.int32),
                       pltpu.VMEM((per_w,), jnp.int32),
                       pltpu.SemaphoreType.DMA],
    )
    def gather_kernel(table_hbm, idx_hbm, out_hbm, idx_v, rows_v, sem):
        wid = lax.axis_index("s") * nc + lax.axis_index("c")

        @pl.when(wid < n_workers)
        def _():
            base = wid * per_w
            pltpu.sync_copy(idx_hbm.at[pl.ds(base, per_w)], idx_v)
            pltpu.async_copy(table_hbm.at[idx_v], rows_v, sem).wait()
            pltpu.sync_copy(rows_v, out_hbm.at[pl.ds(base, per_w)])

    return gather_kernel(action_space, idx)


def kernel(logits, action_space):
    log_probs, idx = _dense_call(logits)
    actions = _sc_gather(action_space, idx)
    return log_probs, actions


# speedup vs baseline: 1.1535x; 1.0134x over previous
import functools

import jax
import jax.numpy as jnp
import numpy as np
from jax import lax
from jax.experimental import pallas as pl
from jax.experimental.pallas import tpu as pltpu
from jax.experimental.pallas import tpu_sc as plsc

B = 128
V = 100000
VC = 1000
NB = V // VC
UNROLL = 4
UNROLL2 = 4

_K1 = np.uint32(0)
_K2 = np.uint32(42)
_TINY = np.float32(1.1754943508222875e-38)
_I32_BIG = np.int32(2**31 - 1)


def _threefry_bits(cnt_lo):
    ks0 = _K1
    ks1 = _K2
    ks2 = np.uint32(_K1 ^ _K2 ^ np.uint32(0x1BD11BDA))

    x0 = jnp.zeros_like(cnt_lo) + ks0
    x1 = cnt_lo + ks1

    def rnd(x0, x1, r):
        x0 = x0 + x1
        x1 = (x1 << np.uint32(r)) | (x1 >> np.uint32(32 - r))
        x1 = x0 ^ x1
        return x0, x1

    for r in (13, 15, 26, 6):
        x0, x1 = rnd(x0, x1, r)
    x0 = x0 + ks1
    x1 = x1 + np.uint32(ks2 + np.uint32(1))
    for r in (17, 29, 16, 24):
        x0, x1 = rnd(x0, x1, r)
    x0 = x0 + ks2
    x1 = x1 + np.uint32(ks0 + np.uint32(2))
    for r in (13, 15, 26, 6):
        x0, x1 = rnd(x0, x1, r)
    x0 = x0 + ks0
    x1 = x1 + np.uint32(ks1 + np.uint32(3))
    for r in (17, 29, 16, 24):
        x0, x1 = rnd(x0, x1, r)
    x0 = x0 + ks1
    x1 = x1 + np.uint32(ks2 + np.uint32(4))
    for r in (13, 15, 26, 6):
        x0, x1 = rnd(x0, x1, r)
    x0 = x0 + ks2
    x1 = x1 + np.uint32(ks0 + np.uint32(5))
    return x0 ^ x1


BR = 8
NR = B // BR


def _dense_body(logits_ref, lp_ref, idx_ref):
    j = pl.program_id(0)
    row0 = j * BR

    row = lax.broadcasted_iota(jnp.int32, (BR, VC), 0) + row0
    col0 = lax.broadcasted_iota(jnp.int32, (BR, VC), 1)
    cnt0 = row * V + col0

    def chunk(k, carry):
        a_score, a_cnt = carry
        blk = logits_ref[:, k, :]
        cnt = cnt0 + k * VC

        bits = _threefry_bits(cnt.astype(jnp.uint32))
        fbits = (bits >> np.uint32(9)) | np.uint32(0x3F800000)
        floats = lax.bitcast_convert_type(fbits, jnp.float32) - np.float32(1.0)
        u = jnp.maximum(_TINY, floats + _TINY)
        score = blk - jnp.log(-jnp.log(u))

        upd = score > a_score
        a_score = jnp.where(upd, score, a_score)
        a_cnt = jnp.where(upd, cnt, a_cnt)
        return a_score, a_cnt

    neg_inf = jnp.full((BR, VC), -jnp.inf, jnp.float32)
    a_score, a_cnt = lax.fori_loop(
        0, NB, chunk, (neg_inf, jnp.zeros((BR, VC), jnp.int32)),
        unroll=UNROLL)

    def chunk2a(k, a_lmax):
        return jnp.maximum(a_lmax, logits_ref[:, k, :])

    a_lmax = lax.fori_loop(0, NB, chunk2a, neg_inf, unroll=UNROLL2)

    def chunk2b(k, se):
        return se + jnp.exp(logits_ref[:, k, :] - a_lmax)

    se = lax.fori_loop(0, NB, chunk2b, jnp.zeros((BR, VC), jnp.float32),
                       unroll=UNROLL2)

    m_row = jnp.max(a_lmax, axis=1)
    se_row = jnp.sum(se * jnp.exp(a_lmax - m_row[:, None]), axis=1)

    m_sc = jnp.max(a_score, axis=1)
    cand = jnp.where(a_score == m_sc[:, None], a_cnt, _I32_BIG)
    cnt_win = jnp.min(cand, axis=1)

    bits_w = _threefry_bits(cnt_win.astype(jnp.uint32))
    fb_w = (bits_w >> np.uint32(9)) | np.uint32(0x3F800000)
    fl_w = lax.bitcast_convert_type(fb_w, jnp.float32) - np.float32(1.0)
    u_w = jnp.maximum(_TINY, fl_w + _TINY)
    l_win = m_sc + jnp.log(-jnp.log(u_w))

    rvec = cnt0[:, 0]
    lp_ref[j, :] = l_win - (m_row + jnp.log(se_row))
    idx_ref[j, :] = cnt_win - rvec


def _dense_call(logits, interpret=False):
    lp2, idx2 = pl.pallas_call(
        _dense_body,
        grid=(NR,),
        in_specs=[pl.BlockSpec((BR, NB, VC), lambda j: (j, 0, 0))],
        out_specs=[pl.BlockSpec((NR, BR), lambda j: (0, 0)),
                   pl.BlockSpec((NR, BR), lambda j: (0, 0))],
        out_shape=[jax.ShapeDtypeStruct((NR, BR), jnp.float32),
                   jax.ShapeDtypeStruct((NR, BR), jnp.int32)],
        compiler_params=pltpu.CompilerParams(
            dimension_semantics=("parallel",)),
        interpret=interpret,
    )(logits.reshape(B, NB, VC))
    return lp2.reshape(B), idx2.reshape(B)


def _sc_gather(action_space, idx):
    info = plsc.get_sparse_core_info()
    nc = info.num_cores
    n_workers = 16
    per_w = B // n_workers
    mesh = plsc.VectorSubcoreMesh(core_axis_name="c", subcore_axis_name="s")

    @functools.partial(
        pl.kernel, mesh=mesh,
        out_type=jax.ShapeDtypeStruct((B,), jnp.int32),
        scratch_types=[pltpu.VMEM((per_w,), jnp.int32),
                       pltpu.VMEM((per_w,), jnp.int32),
                       pltpu.SemaphoreType.DMA],
    )
    def gather_kernel(table_hbm, idx_hbm, out_hbm, idx_v, rows_v, sem):
        wid = lax.axis_index("s") * nc + lax.axis_index("c")

        @pl.when(wid < n_workers)
        def _():
            base = wid * per_w
            pltpu.sync_copy(idx_hbm.at[pl.ds(base, per_w)], idx_v)
            pltpu.async_copy(table_hbm.at[idx_v], rows_v, sem).wait()
            pltpu.sync_copy(rows_v, out_hbm.at[pl.ds(base, per_w)])

    return gather_kernel(action_space, idx)


def kernel(logits, action_space):
    log_probs, idx = _dense_call(logits)
    actions = _sc_gather(action_space, idx)
    return log_probs, actions


# speedup vs baseline: 1.1678x; 1.0124x over previous
import functools

import jax
import jax.numpy as jnp
import numpy as np
from jax import lax
from jax.experimental import pallas as pl
from jax.experimental.pallas import tpu as pltpu
from jax.experimental.pallas import tpu_sc as plsc

B = 128
V = 100000
VC = 1000
NB = V // VC
UNROLL = 10
UNROLL2 = 4

_K1 = np.uint32(0)
_K2 = np.uint32(42)
_TINY = np.float32(1.1754943508222875e-38)
_I32_BIG = np.int32(2**31 - 1)


def _threefry_bits(cnt_lo):
    ks0 = _K1
    ks1 = _K2
    ks2 = np.uint32(_K1 ^ _K2 ^ np.uint32(0x1BD11BDA))

    x0 = jnp.zeros_like(cnt_lo) + ks0
    x1 = cnt_lo + ks1

    def rnd(x0, x1, r):
        x0 = x0 + x1
        x1 = (x1 << np.uint32(r)) | (x1 >> np.uint32(32 - r))
        x1 = x0 ^ x1
        return x0, x1

    for r in (13, 15, 26, 6):
        x0, x1 = rnd(x0, x1, r)
    x0 = x0 + ks1
    x1 = x1 + np.uint32(ks2 + np.uint32(1))
    for r in (17, 29, 16, 24):
        x0, x1 = rnd(x0, x1, r)
    x0 = x0 + ks2
    x1 = x1 + np.uint32(ks0 + np.uint32(2))
    for r in (13, 15, 26, 6):
        x0, x1 = rnd(x0, x1, r)
    x0 = x0 + ks0
    x1 = x1 + np.uint32(ks1 + np.uint32(3))
    for r in (17, 29, 16, 24):
        x0, x1 = rnd(x0, x1, r)
    x0 = x0 + ks1
    x1 = x1 + np.uint32(ks2 + np.uint32(4))
    for r in (13, 15, 26, 6):
        x0, x1 = rnd(x0, x1, r)
    x0 = x0 + ks2
    x1 = x1 + np.uint32(ks0 + np.uint32(5))
    return x0 ^ x1


BR = 8
NR = B // BR


def _dense_body(logits_ref, lp_ref, idx_ref):
    j = pl.program_id(0)
    row0 = j * BR

    row = lax.broadcasted_iota(jnp.int32, (BR, VC), 0) + row0
    col0 = lax.broadcasted_iota(jnp.int32, (BR, VC), 1)
    cnt0 = row * V + col0

    def chunk(k, carry):
        a_score, a_cnt = carry
        blk = logits_ref[:, k, :]
        cnt = cnt0 + k * VC

        bits = _threefry_bits(cnt.astype(jnp.uint32))
        fbits = (bits >> np.uint32(9)) | np.uint32(0x3F800000)
        floats = lax.bitcast_convert_type(fbits, jnp.float32) - np.float32(1.0)
        u = jnp.maximum(_TINY, floats + _TINY)
        score = blk - jnp.log(-jnp.log(u))

        upd = score > a_score
        a_score = jnp.where(upd, score, a_score)
        a_cnt = jnp.where(upd, cnt, a_cnt)
        return a_score, a_cnt

    neg_inf = jnp.full((BR, VC), -jnp.inf, jnp.float32)
    a_score, a_cnt = lax.fori_loop(
        0, NB, chunk, (neg_inf, jnp.zeros((BR, VC), jnp.int32)),
        unroll=UNROLL)

    def chunk2a(k, a_lmax):
        return jnp.maximum(a_lmax, logits_ref[:, k, :])

    a_lmax = lax.fori_loop(0, NB, chunk2a, neg_inf, unroll=UNROLL2)

    def chunk2b(k, se):
        return se + jnp.exp(logits_ref[:, k, :] - a_lmax)

    se = lax.fori_loop(0, NB, chunk2b, jnp.zeros((BR, VC), jnp.float32),
                       unroll=UNROLL2)

    m_row = jnp.max(a_lmax, axis=1)
    se_row = jnp.sum(se * jnp.exp(a_lmax - m_row[:, None]), axis=1)

    m_sc = jnp.max(a_score, axis=1)
    cand = jnp.where(a_score == m_sc[:, None], a_cnt, _I32_BIG)
    cnt_win = jnp.min(cand, axis=1)

    bits_w = _threefry_bits(cnt_win.astype(jnp.uint32))
    fb_w = (bits_w >> np.uint32(9)) | np.uint32(0x3F800000)
    fl_w = lax.bitcast_convert_type(fb_w, jnp.float32) - np.float32(1.0)
    u_w = jnp.maximum(_TINY, fl_w + _TINY)
    l_win = m_sc + jnp.log(-jnp.log(u_w))

    rvec = cnt0[:, 0]
    lp_ref[j, :] = l_win - (m_row + jnp.log(se_row))
    idx_ref[j, :] = cnt_win - rvec


def _dense_call(logits, interpret=False):
    lp2, idx2 = pl.pallas_call(
        _dense_body,
        grid=(NR,),
        in_specs=[pl.BlockSpec((BR, NB, VC), lambda j: (j, 0, 0))],
        out_specs=[pl.BlockSpec((NR, BR), lambda j: (0, 0)),
                   pl.BlockSpec((NR, BR), lambda j: (0, 0))],
        out_shape=[jax.ShapeDtypeStruct((NR, BR), jnp.float32),
                   jax.ShapeDtypeStruct((NR, BR), jnp.int32)],
        compiler_params=pltpu.CompilerParams(
            dimension_semantics=("parallel",)),
        interpret=interpret,
    )(logits.reshape(B, NB, VC))
    return lp2.reshape(B), idx2.reshape(B)


def _sc_gather(action_space, idx):
    info = plsc.get_sparse_core_info()
    nc = info.num_cores
    n_workers = 16
    per_w = B // n_workers
    mesh = plsc.VectorSubcoreMesh(core_axis_name="c", subcore_axis_name="s")

    @functools.partial(
        pl.kernel, mesh=mesh,
        out_type=jax.ShapeDtypeStruct((B,), jnp.int32),
        scratch_types=[pltpu.VMEM((per_w,), jnp.int32),
                       pltpu.VMEM((per_w,), jnp.int32),
                       pltpu.SemaphoreType.DMA],
    )
    def gather_kernel(table_hbm, idx_hbm, out_hbm, idx_v, rows_v, sem):
        wid = lax.axis_index("s") * nc + lax.axis_index("c")

        @pl.when(wid < n_workers)
        def _():
            base = wid * per_w
            pltpu.sync_copy(idx_hbm.at[pl.ds(base, per_w)], idx_v)
            pltpu.async_copy(table_hbm.at[idx_v], rows_v, sem).wait()
            pltpu.sync_copy(rows_v, out_hbm.at[pl.ds(base, per_w)])

    return gather_kernel(action_space, idx)


def kernel(logits, action_space):
    log_probs, idx = _dense_call(logits)
    actions = _sc_gather(action_space, idx)
    return log_probs, actions
